# baseline (speedup 1.0000x reference)
import functools

import jax
import jax.numpy as jnp
import numpy as np
from jax.experimental import pallas as pl
from jax.experimental.pallas import tpu as pltpu

_ALPHA = 4


@functools.lru_cache(maxsize=None)
def _slow_indices(T: int) -> tuple:
    n = T // _ALPHA
    i = np.arange(n - 1, dtype=np.float32) / np.float32(n - 1)
    lo, hi = np.float32(0.0), np.float32(T - 1)
    vals = np.concatenate([lo * (np.float32(1.0) - i) + hi * i, [hi]])
    return tuple(int(v) for v in vals.astype(np.int32))


def _fused_body(in_ref, slow_ref, fast_ref, *, sel):
    t = pl.program_id(1)
    fast_ref[...] = in_ref[...]
    is_sel = functools.reduce(jnp.logical_or, [t == v for v in sel])

    @pl.when(is_sel)
    def _():
        slow_ref[...] = in_ref[...]


def kernel(frames):
    C, T, H, W = frames.shape
    sel = _slow_indices(T)
    S = len(sel)

    def slow_slot(t):
        k = sum([(t >= v).astype(jnp.int32) for v in sel]) - 1
        return jnp.maximum(k, 0)

    blk = (1, 1, H, W)
    slow, fast = pl.pallas_call(
        functools.partial(_fused_body, sel=sel),
        grid=(C, T),
        in_specs=[pl.BlockSpec(blk, lambda c, t: (c, t, 0, 0))],
        out_specs=[
            pl.BlockSpec(blk, lambda c, t: (c, slow_slot(t), 0, 0)),
            pl.BlockSpec(blk, lambda c, t: (c, t, 0, 0)),
        ],
        out_shape=[
            jax.ShapeDtypeStruct((C, S, H, W), frames.dtype),
            jax.ShapeDtypeStruct((C, T, H, W), frames.dtype),
        ],
        compiler_params=pltpu.CompilerParams(
            dimension_semantics=("arbitrary", "arbitrary")
        ),
    )(frames)
    return (slow, fast)
